# baseline (speedup 1.0000x reference)
import functools

import jax
import jax.numpy as jnp
from jax import lax
from jax.experimental import pallas as pl
from jax.experimental.pallas import tpu as pltpu
from jax.experimental.pallas import tpu_sc as plsc

L, B, H, S, V, QK = 4, 1, 12, 2048, 1024, 8
LH = L * H
EPS = 1e-08
NC, NS, LANES = 2, 16, 16
NW = NC * NS
NB = LH - NW
IDXW = 32
CHUNKS = V // LANES


def _gather_body(table, tq_hbm, vis_hbm, out_tgt, out_qsum,
                 tq16_v, idx_a_v, idx_b_v, rows_a, rows_b, vis_v, tgt_v,
                 tgt_v2, acc_v, sem_a, sem_b):
    wid = lax.axis_index("s") * NC + lax.axis_index("c")
    wid_s = wid * S
    b_s = (NW + lax.rem(wid, NS)) * S
    half = jnp.where(wid >= NS, 4, 0)
    pltpu.sync_copy(tq_hbm, tq16_v.at[pl.ds(0, QK)])
    tql = tq16_v[...]
    lane = lax.iota(jnp.int32, 16)
    pos = jnp.where(lane < QK, lane,
                    jnp.where(lane < 12, half + lane - QK, 0))
    vals = tql.at[pos].get(mode="promise_in_bounds")
    base = jnp.where((lane < QK) | (lane == 12), wid_s, b_s)
    idxv = base + jnp.where(lane < 12, vals, S - 1)
    store_scatter = plsc.store_scatter
    store_scatter(idx_a_v, [jnp.where(lane < QK, lane, 0)], idxv,
                  mask=lane < QK)
    store_scatter(idx_b_v, [jnp.clip(lane - QK, 0, QK - 1)], idxv,
                  mask=lane >= QK)
    cp_a = pltpu.make_async_copy(table.at[idx_a_v], rows_a, sem_a)
    cp_b = pltpu.make_async_copy(table.at[idx_b_v], rows_b, sem_b)
    cp_a.start()
    pltpu.sync_copy(vis_hbm, vis_v)

    def col_at(c):
        off = pl.multiple_of(c * LANES, LANES)
        return off, vis_v[pl.ds(off, LANES)]

    def gather_row(rows, j, col):
        row = jnp.full((LANES,), j, jnp.int32)
        return plsc.load_gather(rows, [row, col])

    def qsum_out(slot, qacc):
        acc_v[...] = qacc
        pltpu.sync_copy(acc_v, out_qsum.at[slot])

    zero = jnp.zeros((LANES,), jnp.float32)

    cp_a.wait()
    cp_b.start()

    @plsc.parallel_loop(0, CHUNKS, unroll=4, carry=zero)
    def qacc_a(c, acc):
        _, col = col_at(c)
        g = [gather_row(rows_a, j, col) for j in range(QK)]
        return acc + (((g[0] + g[1]) + (g[2] + g[3]))
                      + ((g[4] + g[5]) + (g[6] + g[7])))

    qsum_out(wid, qacc_a)

    cp_b.wait()

    @plsc.parallel_loop(0, CHUNKS, unroll=4, carry=zero)
    def qacc_b(c, acc):
        off, col = col_at(c)
        g = [gather_row(rows_b, j, col) for j in range(4)]
        tgt_v[pl.ds(off, LANES)] = gather_row(rows_b, 4, col)
        tgt_v2[pl.ds(off, LANES)] = gather_row(rows_b, 5, col)
        return acc + ((g[0] + g[1]) + (g[2] + g[3]))

    qsum_out(NW + wid, qacc_b)
    pltpu.sync_copy(tgt_v, out_tgt.at[wid])

    @pl.when(wid >= NS)
    def _tgt_b():
        pltpu.sync_copy(tgt_v2, out_tgt.at[NW + wid - NS])


_gather = functools.partial(
    pl.kernel,
    out_type=(jax.ShapeDtypeStruct((LH, V), jnp.float32),
              jax.ShapeDtypeStruct((2 * NW, LANES), jnp.float32)),
    mesh=plsc.VectorSubcoreMesh(core_axis_name="c", subcore_axis_name="s"),
    scratch_types=[
        pltpu.VMEM((LANES,), jnp.int32),
        pltpu.VMEM((QK,), jnp.int32),
        pltpu.VMEM((QK,), jnp.int32),
        pltpu.VMEM((QK, S), jnp.float32),
        pltpu.VMEM((QK, S), jnp.float32),
        pltpu.VMEM((V,), jnp.int32),
        pltpu.VMEM((V,), jnp.float32),
        pltpu.VMEM((V,), jnp.float32),
        pltpu.VMEM((LANES,), jnp.float32),
        pltpu.SemaphoreType.DMA,
        pltpu.SemaphoreType.DMA,
    ],
    compiler_params=pltpu.CompilerParams(needs_layout_passes=False),
)(_gather_body)


def _combine_body(qsum16_ref, tgt_ref, labels_ref, out_m_ref, out_l_ref):
    tgt = tgt_ref[...]
    q64 = jnp.sum(qsum16_ref[...], axis=1, keepdims=True)
    q = jnp.concatenate([q64[:NW], q64[NW:NW + NB] + q64[NW + NB:]], axis=0)
    e = jnp.exp(q - jnp.max(q))
    hw = e / jnp.sum(e)
    ii = lax.broadcasted_iota(jnp.int32, (LH, LH), 0)
    jj = lax.broadcasted_iota(jnp.int32, (LH, LH), 1)
    eye = (ii == jj).astype(jnp.float32)
    ones_row = jnp.ones((1, LH), jnp.float32)
    f_col = jnp.sum(tgt, axis=1, keepdims=True)
    f_row = jnp.dot(ones_row, f_col * eye,
                    precision=lax.Precision.HIGHEST)
    cmp = (f_row > f_col) | ((f_row == f_col) & (jj < ii))
    rank = jnp.sum(cmp.astype(jnp.float32), axis=1, keepdims=True)
    hw_row = jnp.dot(ones_row, hw * eye,
                     precision=lax.Precision.HIGHEST)
    w = jnp.sum(jnp.where(rank == jj.astype(jnp.float32), hw_row, 0.0),
                axis=1, keepdims=True)
    merged = jnp.sum(w * tgt, axis=0, keepdims=True)
    merged = merged / (jnp.sum(merged) + EPS)
    lab = labels_ref[...]
    t = lab / (jnp.sum(lab) + EPS)
    pred_log = jnp.log(merged)
    safe_t = jnp.where(t > 0, t, 1.0)
    kl = jnp.where(t > 0, t * (jnp.log(safe_t) - pred_log), 0.0)
    out_m_ref[...] = merged
    out_l_ref[...] = jnp.sum(kl, keepdims=True)


_combine = pl.pallas_call(
    _combine_body,
    out_shape=(jax.ShapeDtypeStruct((1, V), jnp.float32),
               jax.ShapeDtypeStruct((1, 1), jnp.float32)),
)


def kernel(query_indices, visual_indices, target_indices, self_attentions,
           topk_query_indices, global_pattern_per_query, batch_idx, labels):
    table = self_attentions.reshape(L * H * S, S)
    vis = visual_indices.astype(jnp.int32)
    tq = topk_query_indices.astype(jnp.int32)
    tgt, qsum16 = _gather(table, tq, vis)
    merged, loss = _combine(qsum16, tgt, labels.astype(jnp.float32))
    return merged, loss.reshape(())


# speedup vs baseline: 1.0103x; 1.0103x over previous
import functools

import jax
import jax.numpy as jnp
from jax import lax
from jax.experimental import pallas as pl
from jax.experimental.pallas import tpu as pltpu
from jax.experimental.pallas import tpu_sc as plsc

L, B, H, S, V, QK = 4, 1, 12, 2048, 1024, 8
LH = L * H
EPS = 1e-08
NC, NS, LANES = 2, 16, 16
NW = NC * NS
NB = LH - NW
IDXW = 32
CHUNKS = V // LANES


def _gather_body(table, tq_hbm, vis_hbm, out_tgt, out_qsum,
                 tq16_v, idx_a_v, idx_b_v, rows_a, rows_b, vis_v, tgt_v,
                 tgt_v2, acc_v, sem_a, sem_b):
    wid = lax.axis_index("s") * NC + lax.axis_index("c")
    wid_s = wid * S
    b_s = (NW + lax.rem(wid, NS)) * S
    half = jnp.where(wid >= NS, 4, 0)
    pltpu.sync_copy(tq_hbm, tq16_v.at[pl.ds(0, QK)])
    tql = tq16_v[...]
    lane = lax.iota(jnp.int32, 16)
    pos = jnp.where(lane < QK, lane,
                    jnp.where(lane < 12, half + lane - QK, 0))
    vals = tql.at[pos].get(mode="promise_in_bounds")
    base = jnp.where((lane < QK) | (lane == 12), wid_s, b_s)
    idxv = base + jnp.where(lane < 12, vals, S - 1)
    store_scatter = plsc.store_scatter
    store_scatter(idx_a_v, [jnp.where(lane < QK, lane, 0)], idxv,
                  mask=lane < QK)
    store_scatter(idx_b_v, [jnp.clip(lane - QK, 0, QK - 1)], idxv,
                  mask=lane >= QK)
    cp_a = pltpu.make_async_copy(table.at[idx_a_v], rows_a, sem_a)
    cp_b = pltpu.make_async_copy(table.at[idx_b_v], rows_b, sem_b)
    cp_a.start()
    cp_b.start()
    pltpu.sync_copy(vis_hbm, vis_v)

    def col_at(c):
        off = pl.multiple_of(c * LANES, LANES)
        return off, vis_v[pl.ds(off, LANES)]

    def gather_row(rows, j, col):
        row = jnp.full((LANES,), j, jnp.int32)
        return plsc.load_gather(rows, [row, col])

    def qsum_out(slot, qacc):
        acc_v[...] = qacc
        pltpu.sync_copy(acc_v, out_qsum.at[slot])

    zero = jnp.zeros((LANES,), jnp.float32)

    cp_a.wait()

    @plsc.parallel_loop(0, CHUNKS, unroll=4, carry=zero)
    def qacc_a(c, acc):
        _, col = col_at(c)
        g = [gather_row(rows_a, j, col) for j in range(QK)]
        return acc + (((g[0] + g[1]) + (g[2] + g[3]))
                      + ((g[4] + g[5]) + (g[6] + g[7])))

    qsum_out(wid, qacc_a)

    cp_b.wait()

    @plsc.parallel_loop(0, CHUNKS, unroll=4, carry=zero)
    def qacc_b(c, acc):
        off, col = col_at(c)
        g = [gather_row(rows_b, j, col) for j in range(4)]
        tgt_v[pl.ds(off, LANES)] = gather_row(rows_b, 4, col)
        tgt_v2[pl.ds(off, LANES)] = gather_row(rows_b, 5, col)
        return acc + ((g[0] + g[1]) + (g[2] + g[3]))

    qsum_out(NW + wid, qacc_b)
    pltpu.sync_copy(tgt_v, out_tgt.at[wid])

    @pl.when(wid >= NS)
    def _tgt_b():
        pltpu.sync_copy(tgt_v2, out_tgt.at[NW + wid - NS])


_gather = functools.partial(
    pl.kernel,
    out_type=(jax.ShapeDtypeStruct((LH, V), jnp.float32),
              jax.ShapeDtypeStruct((2 * NW, LANES), jnp.float32)),
    mesh=plsc.VectorSubcoreMesh(core_axis_name="c", subcore_axis_name="s"),
    scratch_types=[
        pltpu.VMEM((LANES,), jnp.int32),
        pltpu.VMEM((QK,), jnp.int32),
        pltpu.VMEM((QK,), jnp.int32),
        pltpu.VMEM((QK, S), jnp.float32),
        pltpu.VMEM((QK, S), jnp.float32),
        pltpu.VMEM((V,), jnp.int32),
        pltpu.VMEM((V,), jnp.float32),
        pltpu.VMEM((V,), jnp.float32),
        pltpu.VMEM((LANES,), jnp.float32),
        pltpu.SemaphoreType.DMA,
        pltpu.SemaphoreType.DMA,
    ],
    compiler_params=pltpu.CompilerParams(needs_layout_passes=False),
)(_gather_body)


def _combine_body(qsum16_ref, tgt_ref, labels_ref, out_m_ref, out_l_ref):
    tgt = tgt_ref[...]
    q64 = jnp.sum(qsum16_ref[...], axis=1, keepdims=True)
    q = jnp.concatenate([q64[:NW], q64[NW:NW + NB] + q64[NW + NB:]], axis=0)
    e = jnp.exp(q - jnp.max(q))
    hw = e / jnp.sum(e)
    ii = lax.broadcasted_iota(jnp.int32, (LH, LH), 0)
    jj = lax.broadcasted_iota(jnp.int32, (LH, LH), 1)
    eye = (ii == jj).astype(jnp.float32)
    ones_row = jnp.ones((1, LH), jnp.float32)
    f_col = jnp.sum(tgt, axis=1, keepdims=True)
    f_row = jnp.dot(ones_row, f_col * eye,
                    precision=lax.Precision.HIGHEST)
    cmp = (f_row > f_col) | ((f_row == f_col) & (jj < ii))
    rank = jnp.sum(cmp.astype(jnp.float32), axis=1, keepdims=True)
    hw_row = jnp.dot(ones_row, hw * eye,
                     precision=lax.Precision.HIGHEST)
    w = jnp.sum(jnp.where(rank == jj.astype(jnp.float32), hw_row, 0.0),
                axis=1, keepdims=True)
    merged = jnp.sum(w * tgt, axis=0, keepdims=True)
    merged = merged / (jnp.sum(merged) + EPS)
    lab = labels_ref[...]
    t = lab / (jnp.sum(lab) + EPS)
    pred_log = jnp.log(merged)
    safe_t = jnp.where(t > 0, t, 1.0)
    kl = jnp.where(t > 0, t * (jnp.log(safe_t) - pred_log), 0.0)
    out_m_ref[...] = merged
    out_l_ref[...] = jnp.sum(kl, keepdims=True)


_combine = pl.pallas_call(
    _combine_body,
    out_shape=(jax.ShapeDtypeStruct((1, V), jnp.float32),
               jax.ShapeDtypeStruct((1, 1), jnp.float32)),
)


def kernel(query_indices, visual_indices, target_indices, self_attentions,
           topk_query_indices, global_pattern_per_query, batch_idx, labels):
    table = self_attentions.reshape(L * H * S, S)
    vis = visual_indices.astype(jnp.int32)
    tq = topk_query_indices.astype(jnp.int32)
    tgt, qsum16 = _gather(table, tq, vis)
    merged, loss = _combine(qsum16, tgt, labels.astype(jnp.float32))
    return merged, loss.reshape(())


# speedup vs baseline: 1.0165x; 1.0061x over previous
import functools

import jax
import jax.numpy as jnp
from jax import lax
from jax.experimental import pallas as pl
from jax.experimental.pallas import tpu as pltpu
from jax.experimental.pallas import tpu_sc as plsc

L, B, H, S, V, QK = 4, 1, 12, 2048, 1024, 8
LH = L * H
EPS = 1e-08
NC, NS, LANES = 2, 16, 16
NW = NC * NS
NB = LH - NW
IDXW = 32
CHUNKS = V // LANES


def _gather_body(table, tq_hbm, vis_hbm, out_tgt, out_qsum,
                 tq16_v, idx_a_v, idx_b_v, rows_a, rows_b, vis_v, tgt_v,
                 tgt_v2, acc_v, sem_a, sem_b):
    wid = lax.axis_index("s") * NC + lax.axis_index("c")
    wid_s = wid * S
    b_s = (NW + lax.rem(wid, NS)) * S
    half = jnp.where(wid >= NS, 4, 0)
    pltpu.sync_copy(tq_hbm, tq16_v.at[pl.ds(0, QK)])
    tql = tq16_v[...]
    lane = lax.iota(jnp.int32, 16)
    pos = jnp.where(lane < QK, lane,
                    jnp.where(lane < 12, half + lane - QK, 0))
    vals = tql.at[pos].get(mode="promise_in_bounds")
    base = jnp.where((lane < QK) | (lane == 12), wid_s, b_s)
    idxv = base + jnp.where(lane < 12, vals, S - 1)
    store_scatter = plsc.store_scatter
    store_scatter(idx_a_v, [jnp.where(lane < QK, lane, 0)], idxv,
                  mask=lane < QK)
    store_scatter(idx_b_v, [jnp.clip(lane - QK, 0, QK - 1)], idxv,
                  mask=lane >= QK)
    cp_a = pltpu.make_async_copy(table.at[idx_a_v], rows_a, sem_a)
    cp_b = pltpu.make_async_copy(table.at[idx_b_v], rows_b, sem_b)
    cp_a.start()
    cp_b.start()
    pltpu.sync_copy(vis_hbm, vis_v)

    def col_at(c):
        off = pl.multiple_of(c * LANES, LANES)
        return off, vis_v[pl.ds(off, LANES)]

    def gather_row(rows, j, col):
        row = jnp.full((LANES,), j, jnp.int32)
        return plsc.load_gather(rows, [row, col])

    def qsum_out(slot, qacc):
        acc_v[...] = qacc
        pltpu.sync_copy(acc_v, out_qsum.at[slot])

    zero = jnp.zeros((LANES,), jnp.float32)

    with jax.named_scope("wait_a"):
        cp_a.wait()

    with jax.named_scope("loop_a"):
        @plsc.parallel_loop(0, CHUNKS, unroll=2, carry=zero)
        def qacc_a(c, acc):
            _, col = col_at(c)
            g = [gather_row(rows_a, j, col) for j in range(QK)]
            return acc + (((g[0] + g[1]) + (g[2] + g[3]))
                          + ((g[4] + g[5]) + (g[6] + g[7])))

        qsum_out(wid, qacc_a)

    with jax.named_scope("wait_b"):
        cp_b.wait()

    with jax.named_scope("loop_b"):
        @plsc.parallel_loop(0, CHUNKS, unroll=2, carry=zero)
        def qacc_b(c, acc):
            off, col = col_at(c)
            g = [gather_row(rows_b, j, col) for j in range(4)]
            tgt_v[pl.ds(off, LANES)] = gather_row(rows_b, 4, col)
            tgt_v2[pl.ds(off, LANES)] = gather_row(rows_b, 5, col)
            return acc + ((g[0] + g[1]) + (g[2] + g[3]))

        qsum_out(NW + wid, qacc_b)
        pltpu.sync_copy(tgt_v, out_tgt.at[wid])

    @pl.when(wid >= NS)
    def _tgt_b():
        pltpu.sync_copy(tgt_v2, out_tgt.at[NW + wid - NS])


_gather = functools.partial(
    pl.kernel,
    out_type=(jax.ShapeDtypeStruct((LH, V), jnp.float32),
              jax.ShapeDtypeStruct((2 * NW, LANES), jnp.float32)),
    mesh=plsc.VectorSubcoreMesh(core_axis_name="c", subcore_axis_name="s"),
    scratch_types=[
        pltpu.VMEM((LANES,), jnp.int32),
        pltpu.VMEM((QK,), jnp.int32),
        pltpu.VMEM((QK,), jnp.int32),
        pltpu.VMEM((QK, S), jnp.float32),
        pltpu.VMEM((QK, S), jnp.float32),
        pltpu.VMEM((V,), jnp.int32),
        pltpu.VMEM((V,), jnp.float32),
        pltpu.VMEM((V,), jnp.float32),
        pltpu.VMEM((LANES,), jnp.float32),
        pltpu.SemaphoreType.DMA,
        pltpu.SemaphoreType.DMA,
    ],
    compiler_params=pltpu.CompilerParams(needs_layout_passes=False),
)(_gather_body)


def _combine_body(qsum16_ref, tgt_ref, labels_ref, out_m_ref, out_l_ref):
    tgt = tgt_ref[...]
    q64 = jnp.sum(qsum16_ref[...], axis=1, keepdims=True)
    q = jnp.concatenate([q64[:NW], q64[NW:NW + NB] + q64[NW + NB:]], axis=0)
    e = jnp.exp(q - jnp.max(q))
    hw = e / jnp.sum(e)
    ii = lax.broadcasted_iota(jnp.int32, (LH, LH), 0)
    jj = lax.broadcasted_iota(jnp.int32, (LH, LH), 1)
    eye = (ii == jj).astype(jnp.float32)
    ones_row = jnp.ones((1, LH), jnp.float32)
    f_col = jnp.sum(tgt, axis=1, keepdims=True)
    f_row = jnp.dot(ones_row, f_col * eye,
                    precision=lax.Precision.HIGHEST)
    cmp = (f_row > f_col) | ((f_row == f_col) & (jj < ii))
    rank = jnp.sum(cmp.astype(jnp.float32), axis=1, keepdims=True)
    hw_row = jnp.dot(ones_row, hw * eye,
                     precision=lax.Precision.HIGHEST)
    w = jnp.sum(jnp.where(rank == jj.astype(jnp.float32), hw_row, 0.0),
                axis=1, keepdims=True)
    merged = jnp.sum(w * tgt, axis=0, keepdims=True)
    merged = merged / (jnp.sum(merged) + EPS)
    lab = labels_ref[...]
    t = lab / (jnp.sum(lab) + EPS)
    pred_log = jnp.log(merged)
    safe_t = jnp.where(t > 0, t, 1.0)
    kl = jnp.where(t > 0, t * (jnp.log(safe_t) - pred_log), 0.0)
    out_m_ref[...] = merged
    out_l_ref[...] = jnp.sum(kl, keepdims=True)


_combine = pl.pallas_call(
    _combine_body,
    out_shape=(jax.ShapeDtypeStruct((1, V), jnp.float32),
               jax.ShapeDtypeStruct((1, 1), jnp.float32)),
)


def kernel(query_indices, visual_indices, target_indices, self_attentions,
           topk_query_indices, global_pattern_per_query, batch_idx, labels):
    table = self_attentions.reshape(L * H * S, S)
    vis = visual_indices.astype(jnp.int32)
    tq = topk_query_indices.astype(jnp.int32)
    tgt, qsum16 = _gather(table, tq, vis)
    merged, loss = _combine(qsum16, tgt, labels.astype(jnp.float32))
    return merged, loss.reshape(())
